# baseline (speedup 1.0000x reference)
import functools

import numpy as np
import jax
import jax.numpy as jnp
from jax import lax
from jax.experimental import pallas as pl
from jax.experimental.pallas import tpu as pltpu
from jax.experimental.pallas import tpu_sc as plsc

GAMA_R = 0.2
FILL = -1.0


_ROTS = ((13, 15, 26, 6), (17, 29, 16, 24))
_INJ = ((1, 2, 1), (2, 0, 2), (0, 1, 3), (1, 2, 4), (2, 0, 5))


def _np_rotl(x, r):
    return (x << np.uint32(r)) | (x >> np.uint32(32 - r))


def _np_threefry2x32(k0, k1, x0, x1):
    x0 = np.asarray(x0, np.uint32).copy()
    x1 = np.asarray(x1, np.uint32).copy()
    ks = (np.uint32(k0), np.uint32(k1),
          np.uint32(k0) ^ np.uint32(k1) ^ np.uint32(0x1BD11BDA))
    x0 = x0 + ks[0]
    x1 = x1 + ks[1]
    for i in range(5):
        for r in _ROTS[i % 2]:
            x0 = x0 + x1
            x1 = _np_rotl(x1, r)
            x1 = x1 ^ x0
        a, b, c = _INJ[i]
        x0 = x0 + ks[a]
        x1 = x1 + ks[b] + np.uint32(c)
    return x0, x1


def _derive_constants():
    f0, f1 = _np_threefry2x32(0, 1234, [0], [1])
    k0, k1 = int(f0[0]), int(f1[0])
    u0, u1 = _np_threefry2x32(0, 1234, [0], [0])
    bits = u0 ^ u1
    f = ((bits >> np.uint32(9)) | np.uint32(0x3F800000)).view(np.float32)
    p = np.float32(f[0] - np.float32(1.0)) * np.float32(1.0 - GAMA_R)
    import math
    thresh = np.uint32(math.ceil(float(p) * 8388608.0) << 9)
    return k0, k1, np.float32(p), thresh


_K0, _K1, _P, _BITS_LT = _derive_constants()


def _mask_kernel(mk_ref, keep_ref, *, kk):
    rows, cols = mk_ref.shape
    n = rows * cols
    abits = pltpu.bitcast(mk_ref[...], jnp.int32) & jnp.int32(0x7FFFFFFF)

    def count_ge(v):
        return jnp.sum((abits >= v).astype(jnp.int32))

    def vbody(_, lohi):
        lo, hi = lohi
        mid = (lo + hi) >> 1
        pred = count_ge(mid) >= kk
        return jnp.where(pred, mid, lo), jnp.where(pred, hi, mid)

    t, _ = jax.lax.fori_loop(0, 31, vbody,
                             (jnp.int32(0), jnp.int32(0x7F800001)))

    gt = abits > t
    eq = abits == t
    need_eq = jnp.int32(kk) - jnp.sum(gt.astype(jnp.int32))
    gidx = (jax.lax.broadcasted_iota(jnp.int32, (rows, cols), 0) * cols
            + jax.lax.broadcasted_iota(jnp.int32, (rows, cols), 1))
    eqi = eq.astype(jnp.int32)

    def ibody(_, lohi):
        lo, hi = lohi
        mid = (lo + hi) >> 1
        pred = jnp.sum(jnp.where(gidx < mid, eqi, 0)) >= need_eq
        return jnp.where(pred, lo, mid), jnp.where(pred, mid, hi)

    _, icut = jax.lax.fori_loop(0, 14, ibody, (jnp.int32(0), jnp.int32(n)))

    keep_ref[...] = (gt | (eq & (gidx < icut))).astype(jnp.float32)


def _dropout_kernel(keep_ref, v_ref, out_ref, *, row0=0, d=8192):
    i = pl.program_id(0)
    r, d = v_ref.shape
    base = (row0 * d + i * (r * d)).astype(jnp.uint32)
    row = jax.lax.broadcasted_iota(jnp.uint32, (r, d), 0)
    col = jax.lax.broadcasted_iota(jnp.uint32, (r, d), 1)

    ks0 = np.uint32(_K0)
    ks1 = np.uint32(_K1)
    ks2 = np.uint32(_K0) ^ np.uint32(_K1) ^ np.uint32(0x1BD11BDA)
    ks = (ks0, ks1, ks2)

    x1 = row * np.uint32(d) + col + base + ks1
    x0 = jnp.full((r, d), ks0, dtype=jnp.uint32)
    for blk in range(5):
        for rot in _ROTS[blk % 2]:
            x0 = x0 + x1
            x1 = ((x1 << np.uint32(rot)) | (x1 >> np.uint32(32 - rot))) ^ x0
        a, b, c = _INJ[blk]
        x0 = x0 + ks[a]
        x1 = x1 + ks[b] + np.uint32(c)

    bits = x0 ^ x1
    cond = (bits < _BITS_LT) | (keep_ref[...] > 0.0)
    out_ref[...] = jnp.where(cond, v_ref[...], jnp.float32(FILL))


_KS0 = np.uint32(_K0)
_KS1 = np.uint32(_K1)
_KS2 = np.uint32(_K0) ^ np.uint32(_K1) ^ np.uint32(0x1BD11BDA)
_KS = (_KS0, _KS1, _KS2)


def _tf_keep16(x1):
    x0 = jnp.full((16,), _KS0, dtype=jnp.uint32)
    x1 = x1 + _KS1
    for blk in range(5):
        for rot in _ROTS[blk % 2]:
            x0 = x0 + x1
            x1 = ((x1 << np.uint32(rot)) | (x1 >> np.uint32(32 - rot))) ^ x0
        a, b, c = _INJ[blk]
        x0 = x0 + _KS[a]
        x1 = x1 + _KS[b] + np.uint32(c)
    return (x0 ^ x1) < _BITS_LT


def _sc_bits(keep_hbm, out_hbm, keepbuf, obuf, *, row0, rows_per_worker, d):
    wid = lax.axis_index("s") * 2 + lax.axis_index("c")
    pltpu.sync_copy(keep_hbm, keepbuf)
    iota16 = lax.iota(jnp.int32, 16)
    one16 = jnp.full((16,), 1.0, dtype=jnp.float32)
    zero16 = jnp.full((16,), 0.0, dtype=jnp.float32)

    def row_body(r, _):
        grow = row0 + wid * rows_per_worker + r
        orow = wid * rows_per_worker + r
        jbase = grow * d

        def col_body(cstep, __):
            for u in range(4):
                c0 = cstep * 64 + u * 16
                kf16 = keepbuf[pl.ds(c0, 16)]
                ctr = (jbase + c0 + iota16).astype(jnp.uint32)
                cond = _tf_keep16(ctr) | (kf16 > 0.0)
                obuf[pl.ds(c0, 16)] = jnp.where(cond, one16, zero16)
            return 0

        jax.lax.fori_loop(0, d // 64, col_body, 0)
        pltpu.sync_copy(obuf, out_hbm.at[pl.ds(orow * d, d)])
        return 0

    jax.lax.fori_loop(0, rows_per_worker, row_body, 0)


def _sc_call(keep_flat, row0, nrows, d):
    nworkers = 32
    rpw = nrows // nworkers
    mesh = plsc.VectorSubcoreMesh(core_axis_name="c", subcore_axis_name="s")
    body = functools.partial(_sc_bits, row0=row0,
                             rows_per_worker=rpw, d=d)
    f = functools.partial(
        pl.kernel,
        out_type=jax.ShapeDtypeStruct((nrows * d,), jnp.float32),
        scratch_types=[
            pltpu.VMEM((d,), jnp.float32),
            pltpu.VMEM((d,), jnp.float32),
        ],
        mesh=mesh,
    )(body)
    return f(keep_flat).reshape(nrows, d)


def kernel(variables, model_knowledge):
    b, d = variables.shape
    kk = int(d * GAMA_R)

    mk2 = model_knowledge.reshape(8, d // 8)
    keep_feat = pl.pallas_call(
        lambda mr, o: _mask_kernel(mr, o, kk=kk),
        out_shape=jax.ShapeDtypeStruct((8, d // 8), jnp.float32),
    )(mk2).reshape(1, d)

    rows_per_blk = 128

    def _tc_call(nrows):
        return pl.pallas_call(
            _dropout_kernel,
            grid=(nrows // rows_per_blk,),
            in_specs=[
                pl.BlockSpec((1, d), lambda i: (0, 0)),
                pl.BlockSpec((rows_per_blk, d), lambda i: (i, 0)),
            ],
            out_specs=pl.BlockSpec((rows_per_blk, d), lambda i: (i, 0)),
            out_shape=jax.ShapeDtypeStruct((b, d), jnp.float32),
            compiler_params=pltpu.CompilerParams(
                dimension_semantics=("arbitrary",),
            ),
        )(keep_feat, variables)

    b_tc = 2944
    b_sc = b - b_tc
    dec_sc = _sc_call(keep_feat.reshape(d), b_tc, b_sc, d)
    out_a = _tc_call(b_tc)

    blk0 = b_tc // rows_per_blk

    def _merge_kernel(full_ref, v_ref, dec_ref, out_ref):
        out_ref[...] = jnp.where(dec_ref[...] > 0.0, v_ref[...],
                                 jnp.float32(FILL))

    return pl.pallas_call(
        _merge_kernel,
        grid=(b_sc // rows_per_blk,),
        in_specs=[
            pl.BlockSpec(memory_space=pl.ANY),
            pl.BlockSpec((rows_per_blk, d), lambda i: (i + blk0, 0)),
            pl.BlockSpec((rows_per_blk, d), lambda i: (i, 0)),
        ],
        out_specs=pl.BlockSpec((rows_per_blk, d), lambda i: (i + blk0, 0)),
        out_shape=jax.ShapeDtypeStruct((b, d), jnp.float32),
        input_output_aliases={0: 0},
        compiler_params=pltpu.CompilerParams(
            dimension_semantics=("arbitrary",),
        ),
    )(out_a, variables, dec_sc)


# speedup vs baseline: 1.0382x; 1.0382x over previous
import functools

import numpy as np
import jax
import jax.numpy as jnp
from jax import lax
from jax.experimental import pallas as pl
from jax.experimental.pallas import tpu as pltpu
from jax.experimental.pallas import tpu_sc as plsc

GAMA_R = 0.2
FILL = -1.0


_ROTS = ((13, 15, 26, 6), (17, 29, 16, 24))
_INJ = ((1, 2, 1), (2, 0, 2), (0, 1, 3), (1, 2, 4), (2, 0, 5))


def _np_rotl(x, r):
    return (x << np.uint32(r)) | (x >> np.uint32(32 - r))


def _np_threefry2x32(k0, k1, x0, x1):
    x0 = np.asarray(x0, np.uint32).copy()
    x1 = np.asarray(x1, np.uint32).copy()
    ks = (np.uint32(k0), np.uint32(k1),
          np.uint32(k0) ^ np.uint32(k1) ^ np.uint32(0x1BD11BDA))
    x0 = x0 + ks[0]
    x1 = x1 + ks[1]
    for i in range(5):
        for r in _ROTS[i % 2]:
            x0 = x0 + x1
            x1 = _np_rotl(x1, r)
            x1 = x1 ^ x0
        a, b, c = _INJ[i]
        x0 = x0 + ks[a]
        x1 = x1 + ks[b] + np.uint32(c)
    return x0, x1


def _derive_constants():
    f0, f1 = _np_threefry2x32(0, 1234, [0], [1])
    k0, k1 = int(f0[0]), int(f1[0])
    u0, u1 = _np_threefry2x32(0, 1234, [0], [0])
    bits = u0 ^ u1
    f = ((bits >> np.uint32(9)) | np.uint32(0x3F800000)).view(np.float32)
    p = np.float32(f[0] - np.float32(1.0)) * np.float32(1.0 - GAMA_R)
    import math
    thresh = np.uint32(math.ceil(float(p) * 8388608.0) << 9)
    return k0, k1, np.float32(p), thresh


_K0, _K1, _P, _BITS_LT = _derive_constants()


def _mask_kernel(mk_ref, keep_ref, *, kk):
    rows, cols = mk_ref.shape
    n = rows * cols
    abits = pltpu.bitcast(mk_ref[...], jnp.int32) & jnp.int32(0x7FFFFFFF)

    def count_ge(v):
        return jnp.sum((abits >= v).astype(jnp.int32))

    def vbody(_, lohi):
        lo, hi = lohi
        mid = (lo + hi) >> 1
        pred = count_ge(mid) >= kk
        return jnp.where(pred, mid, lo), jnp.where(pred, hi, mid)

    t, _ = jax.lax.fori_loop(0, 31, vbody,
                             (jnp.int32(0), jnp.int32(0x7F800001)))

    gt = abits > t
    eq = abits == t
    need_eq = jnp.int32(kk) - jnp.sum(gt.astype(jnp.int32))
    gidx = (jax.lax.broadcasted_iota(jnp.int32, (rows, cols), 0) * cols
            + jax.lax.broadcasted_iota(jnp.int32, (rows, cols), 1))
    eqi = eq.astype(jnp.int32)

    def ibody(_, lohi):
        lo, hi = lohi
        mid = (lo + hi) >> 1
        pred = jnp.sum(jnp.where(gidx < mid, eqi, 0)) >= need_eq
        return jnp.where(pred, lo, mid), jnp.where(pred, mid, hi)

    _, icut = jax.lax.fori_loop(0, 14, ibody, (jnp.int32(0), jnp.int32(n)))

    keep_ref[...] = (gt | (eq & (gidx < icut))).astype(jnp.float32)


def _dropout_kernel(keep_ref, v_ref, out_ref, *, row0=0, d=8192):
    i = pl.program_id(0)
    r, d = v_ref.shape
    base = (row0 * d + i * (r * d)).astype(jnp.uint32)
    row = jax.lax.broadcasted_iota(jnp.uint32, (r, d), 0)
    col = jax.lax.broadcasted_iota(jnp.uint32, (r, d), 1)

    ks0 = np.uint32(_K0)
    ks1 = np.uint32(_K1)
    ks2 = np.uint32(_K0) ^ np.uint32(_K1) ^ np.uint32(0x1BD11BDA)
    ks = (ks0, ks1, ks2)

    x1 = row * np.uint32(d) + col + base + ks1
    x0 = jnp.full((r, d), ks0, dtype=jnp.uint32)
    for blk in range(5):
        for rot in _ROTS[blk % 2]:
            x0 = x0 + x1
            x1 = ((x1 << np.uint32(rot)) | (x1 >> np.uint32(32 - rot))) ^ x0
        a, b, c = _INJ[blk]
        x0 = x0 + ks[a]
        x1 = x1 + ks[b] + np.uint32(c)

    bits = x0 ^ x1
    cond = (bits < _BITS_LT) | (keep_ref[...] > 0.0)
    out_ref[...] = jnp.where(cond, v_ref[...], jnp.float32(FILL))


_KS0 = np.uint32(_K0)
_KS1 = np.uint32(_K1)
_KS2 = np.uint32(_K0) ^ np.uint32(_K1) ^ np.uint32(0x1BD11BDA)
_KS = (_KS0, _KS1, _KS2)


def _tf_keep16(x1):
    x0 = jnp.full((16,), _KS0, dtype=jnp.uint32)
    x1 = x1 + _KS1
    for blk in range(5):
        for rot in _ROTS[blk % 2]:
            x0 = x0 + x1
            x1 = ((x1 << np.uint32(rot)) | (x1 >> np.uint32(32 - rot))) ^ x0
        a, b, c = _INJ[blk]
        x0 = x0 + _KS[a]
        x1 = x1 + _KS[b] + np.uint32(c)
    return (x0 ^ x1) < _BITS_LT


def _sc_bits(keep_hbm, out_hbm, keepbuf, obuf, *, row0, rows_per_worker, d):
    wid = lax.axis_index("s") * 2 + lax.axis_index("c")
    pltpu.sync_copy(keep_hbm, keepbuf)
    iota16 = lax.iota(jnp.int32, 16)
    one16 = jnp.full((16,), 1.0, dtype=jnp.float32)
    zero16 = jnp.full((16,), 0.0, dtype=jnp.float32)

    def row_body(r, _):
        grow = row0 + wid * rows_per_worker + r
        orow = wid * rows_per_worker + r
        jbase = grow * d

        def col_body(cstep, __):
            for u in range(4):
                c0 = cstep * 64 + u * 16
                kf16 = keepbuf[pl.ds(c0, 16)]
                ctr = (jbase + c0 + iota16).astype(jnp.uint32)
                cond = _tf_keep16(ctr) | (kf16 > 0.0)
                obuf[pl.ds(c0, 16)] = jnp.where(cond, one16, zero16)
            return 0

        jax.lax.fori_loop(0, d // 64, col_body, 0)
        pltpu.sync_copy(obuf, out_hbm.at[pl.ds(orow * d, d)])
        return 0

    jax.lax.fori_loop(0, rows_per_worker, row_body, 0)


def _sc_call(keep_flat, row0, nrows, d):
    nworkers = 32
    rpw = nrows // nworkers
    mesh = plsc.VectorSubcoreMesh(core_axis_name="c", subcore_axis_name="s")
    body = functools.partial(_sc_bits, row0=row0,
                             rows_per_worker=rpw, d=d)
    f = functools.partial(
        pl.kernel,
        out_type=jax.ShapeDtypeStruct((nrows * d,), jnp.float32),
        scratch_types=[
            pltpu.VMEM((d,), jnp.float32),
            pltpu.VMEM((d,), jnp.float32),
        ],
        mesh=mesh,
    )(body)
    return f(keep_flat).reshape(nrows, d)


def kernel(variables, model_knowledge):
    b, d = variables.shape
    kk = int(d * GAMA_R)

    mk2 = model_knowledge.reshape(8, d // 8)
    keep_feat = pl.pallas_call(
        lambda mr, o: _mask_kernel(mr, o, kk=kk),
        out_shape=jax.ShapeDtypeStruct((8, d // 8), jnp.float32),
    )(mk2).reshape(1, d)

    rows_per_blk = 128

    def _tc_call(nrows):
        return pl.pallas_call(
            _dropout_kernel,
            grid=(nrows // rows_per_blk,),
            in_specs=[
                pl.BlockSpec((1, d), lambda i: (0, 0)),
                pl.BlockSpec((rows_per_blk, d), lambda i: (i, 0)),
            ],
            out_specs=pl.BlockSpec((rows_per_blk, d), lambda i: (i, 0)),
            out_shape=jax.ShapeDtypeStruct((b, d), jnp.float32),
            compiler_params=pltpu.CompilerParams(
                dimension_semantics=("arbitrary",),
            ),
        )(keep_feat, variables)

    b_tc = 3200
    b_sc = b - b_tc
    dec_sc = _sc_call(keep_feat.reshape(d), b_tc, b_sc, d)
    out_a = _tc_call(b_tc)

    blk0 = b_tc // rows_per_blk

    def _merge_kernel(full_ref, v_ref, dec_ref, out_ref):
        out_ref[...] = jnp.where(dec_ref[...] > 0.0, v_ref[...],
                                 jnp.float32(FILL))

    return pl.pallas_call(
        _merge_kernel,
        grid=(b_sc // rows_per_blk,),
        in_specs=[
            pl.BlockSpec(memory_space=pl.ANY),
            pl.BlockSpec((rows_per_blk, d), lambda i: (i + blk0, 0)),
            pl.BlockSpec((rows_per_blk, d), lambda i: (i, 0)),
        ],
        out_specs=pl.BlockSpec((rows_per_blk, d), lambda i: (i + blk0, 0)),
        out_shape=jax.ShapeDtypeStruct((b, d), jnp.float32),
        input_output_aliases={0: 0},
        compiler_params=pltpu.CompilerParams(
            dimension_semantics=("arbitrary",),
        ),
    )(out_a, variables, dec_sc)


# speedup vs baseline: 1.0617x; 1.0227x over previous
import functools

import numpy as np
import jax
import jax.numpy as jnp
from jax import lax
from jax.experimental import pallas as pl
from jax.experimental.pallas import tpu as pltpu
from jax.experimental.pallas import tpu_sc as plsc

GAMA_R = 0.2
FILL = -1.0


_ROTS = ((13, 15, 26, 6), (17, 29, 16, 24))
_INJ = ((1, 2, 1), (2, 0, 2), (0, 1, 3), (1, 2, 4), (2, 0, 5))


def _np_rotl(x, r):
    return (x << np.uint32(r)) | (x >> np.uint32(32 - r))


def _np_threefry2x32(k0, k1, x0, x1):
    x0 = np.asarray(x0, np.uint32).copy()
    x1 = np.asarray(x1, np.uint32).copy()
    ks = (np.uint32(k0), np.uint32(k1),
          np.uint32(k0) ^ np.uint32(k1) ^ np.uint32(0x1BD11BDA))
    x0 = x0 + ks[0]
    x1 = x1 + ks[1]
    for i in range(5):
        for r in _ROTS[i % 2]:
            x0 = x0 + x1
            x1 = _np_rotl(x1, r)
            x1 = x1 ^ x0
        a, b, c = _INJ[i]
        x0 = x0 + ks[a]
        x1 = x1 + ks[b] + np.uint32(c)
    return x0, x1


def _derive_constants():
    f0, f1 = _np_threefry2x32(0, 1234, [0], [1])
    k0, k1 = int(f0[0]), int(f1[0])
    u0, u1 = _np_threefry2x32(0, 1234, [0], [0])
    bits = u0 ^ u1
    f = ((bits >> np.uint32(9)) | np.uint32(0x3F800000)).view(np.float32)
    p = np.float32(f[0] - np.float32(1.0)) * np.float32(1.0 - GAMA_R)
    import math
    thresh = np.uint32(math.ceil(float(p) * 8388608.0) << 9)
    return k0, k1, np.float32(p), thresh


_K0, _K1, _P, _BITS_LT = _derive_constants()


def _mask_kernel(mk_ref, keep_ref, *, kk):
    rows, cols = mk_ref.shape
    n = rows * cols
    abits = pltpu.bitcast(mk_ref[...], jnp.int32) & jnp.int32(0x7FFFFFFF)

    def count_ge(v):
        return jnp.sum((abits >= v).astype(jnp.int32))

    def vbody(_, lohi):
        lo, hi = lohi
        mid = (lo + hi) >> 1
        pred = count_ge(mid) >= kk
        return jnp.where(pred, mid, lo), jnp.where(pred, hi, mid)

    t, _ = jax.lax.fori_loop(0, 31, vbody,
                             (jnp.int32(0), jnp.int32(0x7F800001)))

    gt = abits > t
    eq = abits == t
    need_eq = jnp.int32(kk) - jnp.sum(gt.astype(jnp.int32))
    gidx = (jax.lax.broadcasted_iota(jnp.int32, (rows, cols), 0) * cols
            + jax.lax.broadcasted_iota(jnp.int32, (rows, cols), 1))
    eqi = eq.astype(jnp.int32)

    def ibody(_, lohi):
        lo, hi = lohi
        mid = (lo + hi) >> 1
        pred = jnp.sum(jnp.where(gidx < mid, eqi, 0)) >= need_eq
        return jnp.where(pred, lo, mid), jnp.where(pred, mid, hi)

    _, icut = jax.lax.fori_loop(0, 14, ibody, (jnp.int32(0), jnp.int32(n)))

    keep_ref[...] = (gt | (eq & (gidx < icut))).astype(jnp.float32)


def _dropout_kernel(keep_ref, v_ref, out_ref, *, row0=0, d=8192):
    i = pl.program_id(0)
    r, d = v_ref.shape
    base = (row0 * d + i * (r * d)).astype(jnp.uint32)
    row = jax.lax.broadcasted_iota(jnp.uint32, (r, d), 0)
    col = jax.lax.broadcasted_iota(jnp.uint32, (r, d), 1)

    ks0 = np.uint32(_K0)
    ks1 = np.uint32(_K1)
    ks2 = np.uint32(_K0) ^ np.uint32(_K1) ^ np.uint32(0x1BD11BDA)
    ks = (ks0, ks1, ks2)

    x1 = row * np.uint32(d) + col + base + ks1
    x0 = jnp.full((r, d), ks0, dtype=jnp.uint32)
    for blk in range(5):
        for rot in _ROTS[blk % 2]:
            x0 = x0 + x1
            x1 = ((x1 << np.uint32(rot)) | (x1 >> np.uint32(32 - rot))) ^ x0
        a, b, c = _INJ[blk]
        x0 = x0 + ks[a]
        x1 = x1 + ks[b] + np.uint32(c)

    bits = x0 ^ x1
    cond = (bits < _BITS_LT) | (keep_ref[...] > 0.0)
    out_ref[...] = jnp.where(cond, v_ref[...], jnp.float32(FILL))


_KS0 = np.uint32(_K0)
_KS1 = np.uint32(_K1)
_KS2 = np.uint32(_K0) ^ np.uint32(_K1) ^ np.uint32(0x1BD11BDA)
_KS = (_KS0, _KS1, _KS2)


def _tf_keep16(x1):
    x0 = jnp.full((16,), _KS0, dtype=jnp.uint32)
    x1 = x1 + _KS1
    for blk in range(5):
        for rot in _ROTS[blk % 2]:
            x0 = x0 + x1
            x1 = ((x1 << np.uint32(rot)) | (x1 >> np.uint32(32 - rot))) ^ x0
        a, b, c = _INJ[blk]
        x0 = x0 + _KS[a]
        x1 = x1 + _KS[b] + np.uint32(c)
    return (x0 ^ x1) < _BITS_LT


def _sc_bits(keep_hbm, out_hbm, keepbuf, obuf, *, row0, rows_per_worker, d):
    wid = lax.axis_index("s") * 2 + lax.axis_index("c")
    pltpu.sync_copy(keep_hbm, keepbuf)
    iota16 = lax.iota(jnp.int32, 16)
    one16 = jnp.full((16,), 1.0, dtype=jnp.float32)
    zero16 = jnp.full((16,), 0.0, dtype=jnp.float32)

    def row_body(r, _):
        grow = row0 + wid * rows_per_worker + r
        orow = wid * rows_per_worker + r
        jbase = grow * d

        def col_body(cstep, __):
            for u in range(8):
                c0 = cstep * 128 + u * 16
                kf16 = keepbuf[pl.ds(c0, 16)]
                ctr = (jbase + c0 + iota16).astype(jnp.uint32)
                cond = _tf_keep16(ctr) | (kf16 > 0.0)
                obuf[pl.ds(c0, 16)] = jnp.where(cond, one16, zero16)
            return 0

        jax.lax.fori_loop(0, d // 128, col_body, 0)
        pltpu.sync_copy(obuf, out_hbm.at[pl.ds(orow * d, d)])
        return 0

    jax.lax.fori_loop(0, rows_per_worker, row_body, 0)


def _sc_call(keep_flat, row0, nrows, d):
    nworkers = 32
    rpw = nrows // nworkers
    mesh = plsc.VectorSubcoreMesh(core_axis_name="c", subcore_axis_name="s")
    body = functools.partial(_sc_bits, row0=row0,
                             rows_per_worker=rpw, d=d)
    f = functools.partial(
        pl.kernel,
        out_type=jax.ShapeDtypeStruct((nrows * d,), jnp.float32),
        scratch_types=[
            pltpu.VMEM((d,), jnp.float32),
            pltpu.VMEM((d,), jnp.float32),
        ],
        mesh=mesh,
    )(body)
    return f(keep_flat).reshape(nrows, d)


def kernel(variables, model_knowledge):
    b, d = variables.shape
    kk = int(d * GAMA_R)

    mk2 = model_knowledge.reshape(8, d // 8)
    keep_feat = pl.pallas_call(
        lambda mr, o: _mask_kernel(mr, o, kk=kk),
        out_shape=jax.ShapeDtypeStruct((8, d // 8), jnp.float32),
    )(mk2).reshape(1, d)

    rows_per_blk = 128

    def _tc_call(nrows):
        return pl.pallas_call(
            _dropout_kernel,
            grid=(nrows // rows_per_blk,),
            in_specs=[
                pl.BlockSpec((1, d), lambda i: (0, 0)),
                pl.BlockSpec((rows_per_blk, d), lambda i: (i, 0)),
            ],
            out_specs=pl.BlockSpec((rows_per_blk, d), lambda i: (i, 0)),
            out_shape=jax.ShapeDtypeStruct((b, d), jnp.float32),
            compiler_params=pltpu.CompilerParams(
                dimension_semantics=("arbitrary",),
            ),
        )(keep_feat, variables)

    b_tc = 3072
    b_sc = b - b_tc
    dec_sc = _sc_call(keep_feat.reshape(d), b_tc, b_sc, d)
    out_a = _tc_call(b_tc)

    mrows = 256
    mblk0 = b_tc // mrows

    def _merge_kernel(full_ref, v_ref, dec_ref, out_ref):
        out_ref[...] = jnp.where(dec_ref[...] > 0.0, v_ref[...],
                                 jnp.float32(FILL))

    return pl.pallas_call(
        _merge_kernel,
        grid=(b_sc // mrows,),
        in_specs=[
            pl.BlockSpec(memory_space=pl.ANY),
            pl.BlockSpec((mrows, d), lambda i: (i + mblk0, 0)),
            pl.BlockSpec((mrows, d), lambda i: (i, 0)),
        ],
        out_specs=pl.BlockSpec((mrows, d), lambda i: (i + mblk0, 0)),
        out_shape=jax.ShapeDtypeStruct((b, d), jnp.float32),
        input_output_aliases={0: 0},
        compiler_params=pltpu.CompilerParams(
            dimension_semantics=("arbitrary",),
        ),
    )(out_a, variables, dec_sc)


# speedup vs baseline: 1.0715x; 1.0092x over previous
import functools

import numpy as np
import jax
import jax.numpy as jnp
from jax import lax
from jax.experimental import pallas as pl
from jax.experimental.pallas import tpu as pltpu
from jax.experimental.pallas import tpu_sc as plsc

GAMA_R = 0.2
FILL = -1.0


_ROTS = ((13, 15, 26, 6), (17, 29, 16, 24))
_INJ = ((1, 2, 1), (2, 0, 2), (0, 1, 3), (1, 2, 4), (2, 0, 5))


def _np_rotl(x, r):
    return (x << np.uint32(r)) | (x >> np.uint32(32 - r))


def _np_threefry2x32(k0, k1, x0, x1):
    x0 = np.asarray(x0, np.uint32).copy()
    x1 = np.asarray(x1, np.uint32).copy()
    ks = (np.uint32(k0), np.uint32(k1),
          np.uint32(k0) ^ np.uint32(k1) ^ np.uint32(0x1BD11BDA))
    x0 = x0 + ks[0]
    x1 = x1 + ks[1]
    for i in range(5):
        for r in _ROTS[i % 2]:
            x0 = x0 + x1
            x1 = _np_rotl(x1, r)
            x1 = x1 ^ x0
        a, b, c = _INJ[i]
        x0 = x0 + ks[a]
        x1 = x1 + ks[b] + np.uint32(c)
    return x0, x1


def _derive_constants():
    f0, f1 = _np_threefry2x32(0, 1234, [0], [1])
    k0, k1 = int(f0[0]), int(f1[0])
    u0, u1 = _np_threefry2x32(0, 1234, [0], [0])
    bits = u0 ^ u1
    f = ((bits >> np.uint32(9)) | np.uint32(0x3F800000)).view(np.float32)
    p = np.float32(f[0] - np.float32(1.0)) * np.float32(1.0 - GAMA_R)
    import math
    thresh = np.uint32(math.ceil(float(p) * 8388608.0) << 9)
    return k0, k1, np.float32(p), thresh


_K0, _K1, _P, _BITS_LT = _derive_constants()


def _mask_kernel(mk_ref, keep_ref, *, kk):
    rows, cols = mk_ref.shape
    n = rows * cols
    abits = pltpu.bitcast(mk_ref[...], jnp.int32) & jnp.int32(0x7FFFFFFF)

    def count_ge(v):
        return jnp.sum((abits >= v).astype(jnp.int32))

    def vbody(_, lohi):
        lo, hi = lohi
        mid = (lo + hi) >> 1
        pred = count_ge(mid) >= kk
        return jnp.where(pred, mid, lo), jnp.where(pred, hi, mid)

    t, _ = jax.lax.fori_loop(0, 31, vbody,
                             (jnp.int32(0), jnp.int32(0x7F800001)))

    gt = abits > t
    eq = abits == t
    need_eq = jnp.int32(kk) - jnp.sum(gt.astype(jnp.int32))
    gidx = (jax.lax.broadcasted_iota(jnp.int32, (rows, cols), 0) * cols
            + jax.lax.broadcasted_iota(jnp.int32, (rows, cols), 1))
    eqi = eq.astype(jnp.int32)

    def ibody(_, lohi):
        lo, hi = lohi
        mid = (lo + hi) >> 1
        pred = jnp.sum(jnp.where(gidx < mid, eqi, 0)) >= need_eq
        return jnp.where(pred, lo, mid), jnp.where(pred, mid, hi)

    _, icut = jax.lax.fori_loop(0, 14, ibody, (jnp.int32(0), jnp.int32(n)))

    keep_ref[...] = (gt | (eq & (gidx < icut))).astype(jnp.float32)


def _dropout_kernel(keep_ref, xinit_ref, v_ref, out_ref):
    i = pl.program_id(0)
    r, d = v_ref.shape
    base = (i * (r * d)).astype(jnp.uint32)

    ks0 = np.uint32(_K0)
    ks1 = np.uint32(_K1)
    ks2 = np.uint32(_K0) ^ np.uint32(_K1) ^ np.uint32(0x1BD11BDA)
    ks = (ks0, ks1, ks2)

    x1 = xinit_ref[...] + (base + ks1)
    x0 = jnp.full((r, d), ks0, dtype=jnp.uint32)
    for blk in range(5):
        for rot in _ROTS[blk % 2]:
            x0 = x0 + x1
            x1 = ((x1 << np.uint32(rot)) | (x1 >> np.uint32(32 - rot))) ^ x0
        a, b, c = _INJ[blk]
        x0 = x0 + ks[a]
        x1 = x1 + ks[b] + np.uint32(c)

    bits = x0 ^ x1
    cond = (bits < _BITS_LT) | (keep_ref[...] > 0.0)
    out_ref[...] = jnp.where(cond, v_ref[...], jnp.float32(FILL))


_KS0 = np.uint32(_K0)
_KS1 = np.uint32(_K1)
_KS2 = np.uint32(_K0) ^ np.uint32(_K1) ^ np.uint32(0x1BD11BDA)
_KS = (_KS0, _KS1, _KS2)


def _tf_keep16(x1):
    x0 = jnp.full((16,), _KS0, dtype=jnp.uint32)
    x1 = x1 + _KS1
    for blk in range(5):
        for rot in _ROTS[blk % 2]:
            x0 = x0 + x1
            x1 = ((x1 << np.uint32(rot)) | (x1 >> np.uint32(32 - rot))) ^ x0
        a, b, c = _INJ[blk]
        x0 = x0 + _KS[a]
        x1 = x1 + _KS[b] + np.uint32(c)
    return (x0 ^ x1) < _BITS_LT


def _sc_bits(keep_hbm, out_hbm, keepbuf, obuf, *, row0, rows_per_worker, d):
    wid = lax.axis_index("s") * 2 + lax.axis_index("c")
    pltpu.sync_copy(keep_hbm, keepbuf)
    iota16 = lax.iota(jnp.int32, 16)
    one16 = jnp.full((16,), 1.0, dtype=jnp.float32)
    zero16 = jnp.full((16,), 0.0, dtype=jnp.float32)

    def row_body(r, _):
        grow = row0 + wid * rows_per_worker + r
        orow = wid * rows_per_worker + r
        jbase = grow * d

        def col_body(cstep, __):
            for u in range(8):
                c0 = cstep * 128 + u * 16
                kf16 = keepbuf[pl.ds(c0, 16)]
                ctr = (jbase + c0 + iota16).astype(jnp.uint32)
                cond = _tf_keep16(ctr) | (kf16 > 0.0)
                obuf[pl.ds(c0, 16)] = jnp.where(cond, one16, zero16)
            return 0

        jax.lax.fori_loop(0, d // 128, col_body, 0)
        pltpu.sync_copy(obuf, out_hbm.at[pl.ds(orow * d, d)])
        return 0

    jax.lax.fori_loop(0, rows_per_worker, row_body, 0)


def _sc_call(keep_flat, row0, nrows, d):
    nworkers = 32
    rpw = nrows // nworkers
    mesh = plsc.VectorSubcoreMesh(core_axis_name="c", subcore_axis_name="s")
    body = functools.partial(_sc_bits, row0=row0,
                             rows_per_worker=rpw, d=d)
    f = functools.partial(
        pl.kernel,
        out_type=jax.ShapeDtypeStruct((nrows * d,), jnp.float32),
        scratch_types=[
            pltpu.VMEM((d,), jnp.float32),
            pltpu.VMEM((d,), jnp.float32),
        ],
        mesh=mesh,
    )(body)
    return f(keep_flat).reshape(nrows, d)


def kernel(variables, model_knowledge):
    b, d = variables.shape
    kk = int(d * GAMA_R)

    mk2 = model_knowledge.reshape(8, d // 8)
    keep_feat = pl.pallas_call(
        lambda mr, o: _mask_kernel(mr, o, kk=kk),
        out_shape=jax.ShapeDtypeStruct((8, d // 8), jnp.float32),
    )(mk2).reshape(1, d)

    rows_per_blk = 128

    xinit = jnp.asarray(
        np.arange(rows_per_blk * d, dtype=np.uint32).reshape(rows_per_blk, d))

    def _tc_call(nrows):
        return pl.pallas_call(
            _dropout_kernel,
            grid=(nrows // rows_per_blk,),
            in_specs=[
                pl.BlockSpec((1, d), lambda i: (0, 0)),
                pl.BlockSpec((rows_per_blk, d), lambda i: (0, 0)),
                pl.BlockSpec((rows_per_blk, d), lambda i: (i, 0)),
            ],
            out_specs=pl.BlockSpec((rows_per_blk, d), lambda i: (i, 0)),
            out_shape=jax.ShapeDtypeStruct((b, d), jnp.float32),
            compiler_params=pltpu.CompilerParams(
                dimension_semantics=("arbitrary",),
            ),
        )(keep_feat, xinit, variables)

    b_tc = 3072
    b_sc = b - b_tc
    dec_sc = _sc_call(keep_feat.reshape(d), b_tc, b_sc, d)
    out_a = _tc_call(b_tc)

    mrows = 256
    mblk0 = b_tc // mrows

    def _merge_kernel(full_ref, v_ref, dec_ref, out_ref):
        out_ref[...] = jnp.where(dec_ref[...] > 0.0, v_ref[...],
                                 jnp.float32(FILL))

    return pl.pallas_call(
        _merge_kernel,
        grid=(b_sc // mrows,),
        in_specs=[
            pl.BlockSpec(memory_space=pl.ANY),
            pl.BlockSpec((mrows, d), lambda i: (i + mblk0, 0)),
            pl.BlockSpec((mrows, d), lambda i: (i, 0)),
        ],
        out_specs=pl.BlockSpec((mrows, d), lambda i: (i + mblk0, 0)),
        out_shape=jax.ShapeDtypeStruct((b, d), jnp.float32),
        input_output_aliases={0: 0},
        compiler_params=pltpu.CompilerParams(
            dimension_semantics=("arbitrary",),
        ),
    )(out_a, variables, dec_sc)
